# Initial kernel scaffold; baseline (speedup 1.0000x reference)
import jax
import jax.numpy as jnp
from jax.experimental import pallas as pl


def kernel(species, positions, batch, edge_index, compute_forces, species_emb, e_ref, W1, b1, W2, b2, Wsh, Wu, bu, he_W1, he_b1, he_W2, he_b2, he_W3, he_b3):
    raise NotImplementedError("write your pallas kernel here")


# speedup vs baseline: 1.2464x; 1.2464x over previous
import functools

import jax
import jax.numpy as jnp
from jax import lax


def _mm(a, b):
    return jax.lax.dot_general(
        a, b, (((1,), (0,)), ((), ())),
        precision=jax.lax.Precision.HIGHEST,
        preferred_element_type=jnp.float32)
from jax.experimental import pallas as pl
from jax.experimental.pallas import tpu as pltpu
from jax.experimental.pallas import tpu_sc as plsc

NA = 10000
NE = 320000
EP = 327680
DF = 128
NBAS = 8
NBLOCK = 4
NHID = 64
CUTOFF = 5.0
AVGNEIGH = 32.0
NSPEC = 100

NCORE = 2
NSUB = 16
NW = NCORE * NSUB

CHUNK = 1024
KSUB = 8
QUART = 128
PER_TILE = EP // NW
NCHUNK = PER_TILE // CHUNK
OUT_ROWS = 632
OUT_ROWS_LAST = NA - 15 * OUT_ROWS
AGG_ROWS = 10016


@functools.cache
def _mesh():
    return plsc.VectorSubcoreMesh(
        core_axis_name="c", subcore_axis_name="s",
        num_cores=NCORE, num_subcores=NSUB)


GEO_CHUNK = 1024
GEO_NCHUNK = PER_TILE // GEO_CHUNK


def _geo_body(posr_hbm, srcr, dstr, ps_out, pd_out, sidx2, didx2, psbuf,
              pdbuf, sem):
    c = lax.axis_index("c")
    s = lax.axis_index("s")
    wid = c * NSUB + s

    def chunk_body(i, carry):
        base = pl.multiple_of(wid * PER_TILE + i * GEO_CHUNK, GEO_CHUNK)
        brow = pl.multiple_of(base // 128, 8)
        pltpu.async_copy(srcr.at[pl.ds(brow, KSUB)], sidx2, sem).wait()
        pltpu.async_copy(dstr.at[pl.ds(brow, KSUB)], didx2, sem).wait()
        descs = []
        for j in range(KSUB):
            descs.append(pltpu.async_copy(
                posr_hbm.at[sidx2.at[j]], psbuf.at[pl.ds(j * 128, 128)], sem))
            descs.append(pltpu.async_copy(
                posr_hbm.at[didx2.at[j]], pdbuf.at[pl.ds(j * 128, 128)], sem))
        for d in descs:
            d.wait()
        pltpu.async_copy(psbuf, ps_out.at[pl.ds(base, GEO_CHUNK)], sem).wait()
        pltpu.async_copy(pdbuf, pd_out.at[pl.ds(base, GEO_CHUNK)], sem).wait()
        return carry

    lax.fori_loop(0, GEO_NCHUNK, chunk_body, 0)


def _geo_kernel(*args):
    return pl.kernel(
        _geo_body,
        out_type=(jax.ShapeDtypeStruct((EP, 16), jnp.float32),
                  jax.ShapeDtypeStruct((EP, 16), jnp.float32)),
        mesh=_mesh(),
        scratch_types=[
            pltpu.VMEM((KSUB, 128), jnp.int32),
            pltpu.VMEM((KSUB, 128), jnp.int32),
            pltpu.VMEM((GEO_CHUNK, 16), jnp.float32),
            pltpu.VMEM((GEO_CHUNK, 16), jnp.float32),
            pltpu.SemaphoreType.DMA,
        ],
        compiler_params=pltpu.CompilerParams(use_tc_tiling_on_sc=False),
    )(*args)


def _scatter_body(h, gh, srcr, dstr, out, agg_sh, sidx2, didx2, rows, gbuf,
                  sem):
    c = lax.axis_index("c")
    s = lax.axis_index("s")
    wid = c * NSUB + s
    zeros16 = jnp.zeros((16,), jnp.float32)

    def zbody(r, carry):
        for k in range(8):
            rows[r, pl.ds(k * 16, 16)] = zeros16
        return carry
    lax.fori_loop(0, QUART, zbody, 0)
    z0 = s * OUT_ROWS
    for t in range(4):
        pltpu.sync_copy(rows, agg_sh.at[pl.ds(z0 + t * QUART, QUART)])

    @pl.when(s < 15)
    def _():
        pltpu.sync_copy(rows.at[pl.ds(0, 120)],
                        agg_sh.at[pl.ds(z0 + 4 * QUART, 120)])

    @pl.when(s == 15)
    def _():
        pltpu.sync_copy(rows.at[pl.ds(0, 24)],
                        agg_sh.at[pl.ds(z0 + 4 * QUART, 24)])
    plsc.subcore_barrier()

    def chunk_body(i, carry):
        base = pl.multiple_of(wid * PER_TILE + i * CHUNK, CHUNK)
        brow = pl.multiple_of(base // 128, 8)
        pltpu.async_copy(srcr.at[pl.ds(brow, KSUB)], sidx2, sem).wait()
        pltpu.async_copy(dstr.at[pl.ds(brow, KSUB)], didx2, sem).wait()
        for q in range(KSUB):
            qb = pl.multiple_of(base + q * QUART, QUART)
            pltpu.sync_copy(gh.at[pl.ds(qb, QUART)], gbuf)
            pltpu.async_copy(h.at[sidx2.at[q]], rows, sem).wait()

            def mul_body(r, carry2):
                for k in range(8):
                    sl = pl.ds(k * 16, 16)
                    rows[r, sl] = rows[r, sl] * gbuf[r, sl]
                return carry2
            lax.fori_loop(0, QUART, mul_body, 0)

            pltpu.sync_copy(rows, agg_sh.at[didx2.at[q]], add=True)
        return carry

    lax.fori_loop(0, NCHUNK, chunk_body, 0)
    plsc.subcore_barrier()
    o0 = pl.multiple_of(s * OUT_ROWS, 8)
    oo = pl.multiple_of(c * NA + o0, 8)

    @pl.when(s < 15)
    def _():
        pltpu.sync_copy(agg_sh.at[pl.ds(o0, OUT_ROWS)],
                        out.at[pl.ds(oo, OUT_ROWS)])

    @pl.when(s == 15)
    def _():
        pltpu.sync_copy(agg_sh.at[pl.ds(o0, OUT_ROWS_LAST)],
                        out.at[pl.ds(oo, OUT_ROWS_LAST)])


def _scatter_kernel(*args):
    return pl.kernel(
        _scatter_body,
        out_type=jax.ShapeDtypeStruct((2 * NA, DF), jnp.float32),
        mesh=_mesh(),
        scratch_types=[
            pltpu.VMEM_SHARED((AGG_ROWS, DF), jnp.float32),
            pltpu.VMEM((KSUB, 128), jnp.int32),
            pltpu.VMEM((KSUB, 128), jnp.int32),
            pltpu.VMEM((QUART, DF), jnp.float32),
            pltpu.VMEM((QUART, DF), jnp.float32),
            pltpu.SemaphoreType.DMA,
        ],
    )(*args)


GTILE = 1024


def _gains_body(ps_ref, pd_ref, w1a_ref, w2a_ref, wsha_ref, out_ref):
    ps = ps_ref[...]
    pd = pd_ref[...]
    dx = pd[:, 0:1] - ps[:, 0:1]
    dy = pd[:, 1:2] - ps[:, 1:2]
    dz = pd[:, 2:3] - ps[:, 2:3]
    r2 = dx * dx + dy * dy + dz * dz
    r = jnp.sqrt(r2)
    rinv = 1.0 / (r + 1e-8)
    x = dx * rinv
    y = dy * rinv
    z = dz * rinv
    s3 = jnp.sqrt(3.0)
    s15 = jnp.sqrt(15.0)
    s5 = jnp.sqrt(5.0)
    one = jnp.ones_like(x)
    sh = jnp.concatenate([
        one, s3 * x, s3 * y, s3 * z,
        s15 * x * y, s15 * y * z,
        (s5 / 2.0) * (2.0 * z * z - x * x - y * y),
        s15 * x * z, (s15 / 2.0) * (x * x - y * y),
    ], axis=1)
    r_safe = jnp.where(r > 1e-8, r, 1e-8)
    n = (jnp.arange(NBAS, dtype=jnp.int32) + 1).astype(jnp.float32)[None, :]
    rbf = jnp.sqrt(2.0 / CUTOFF) * jnp.sin(n * (jnp.pi / CUTOFF) * r_safe) / r_safe
    u = r / CUTOFF
    u6 = u * u * u * u * u * u
    env = (1.0 - 28.0 * u6 + 48.0 * u6 * u - 21.0 * u6 * u * u)
    env = env * (u < 1.0).astype(jnp.float32)
    rbf = rbf * env
    zpad = jnp.zeros((rbf.shape[0], 119), jnp.float32)
    rbf_aug = jnp.concatenate([rbf, one, zpad], axis=1)
    hid = jax.nn.silu(_mm(rbf_aug, w1a_ref[...]))
    hid_aug = jnp.concatenate(
        [hid, one, jnp.zeros((rbf.shape[0], 127), jnp.float32)], axis=1)
    w = _mm(hid_aug, w2a_ref[...])
    gate = jax.nn.silu(_mm(jnp.concatenate([sh, zpad], axis=1),
                           wsha_ref[...]))
    g = w * gate
    for b in range(NBLOCK):
        out_ref[b] = g[:, b * DF:(b + 1) * DF]


def _gains_call(ps, pd, w1a, w2a, wsha):
    return pl.pallas_call(
        _gains_body,
        grid=(EP // GTILE,),
        in_specs=[
            pl.BlockSpec((GTILE, 16), lambda i: (i, 0)),
            pl.BlockSpec((GTILE, 16), lambda i: (i, 0)),
            pl.BlockSpec((128, 256), lambda i: (0, 0)),
            pl.BlockSpec((384, 512), lambda i: (0, 0)),
            pl.BlockSpec((128, 512), lambda i: (0, 0)),
        ],
        out_specs=pl.BlockSpec((NBLOCK, GTILE, DF), lambda i: (0, i, 0)),
        out_shape=jax.ShapeDtypeStruct((NBLOCK, EP, DF), jnp.float32),
    )(ps, pd, w1a, w2a, wsha)


ATILE = 2000


def _h0_body(sp_ref, emb_ref, out_ref):
    sp = sp_ref[...]
    oh = (sp == jnp.arange(NSPEC, dtype=jnp.int32)[None, :]).astype(jnp.float32)
    out_ref[...] = _mm(oh, emb_ref[...])


def _h0_call(species2, species_emb):
    return pl.pallas_call(
        _h0_body,
        grid=(NA // ATILE,),
        in_specs=[
            pl.BlockSpec((ATILE, 1), lambda i: (i, 0)),
            pl.BlockSpec((NSPEC, DF), lambda i: (0, 0)),
        ],
        out_specs=pl.BlockSpec((ATILE, DF), lambda i: (i, 0)),
        out_shape=jax.ShapeDtypeStruct((NA, DF), jnp.float32),
    )(species2, species_emb)


def _upd_body(h_ref, agg_ref, wua_ref, out_ref):
    a2 = agg_ref[...]
    afull = a2[0] + a2[1]
    one = jnp.ones_like(afull[:, :1])
    zpad = jnp.zeros((afull.shape[0], 127), jnp.float32)
    aug = jnp.concatenate([afull, one, zpad], axis=1)
    out_ref[...] = h_ref[...] + jax.nn.silu(_mm(aug, wua_ref[...]))


def _upd_call(h, agg, wua):
    return pl.pallas_call(
        _upd_body,
        grid=(NA // ATILE,),
        in_specs=[
            pl.BlockSpec((ATILE, DF), lambda i: (i, 0)),
            pl.BlockSpec((2, ATILE, DF), lambda i: (0, i, 0)),
            pl.BlockSpec((2 * DF, DF), lambda i: (0, 0)),
        ],
        out_specs=pl.BlockSpec((ATILE, DF), lambda i: (i, 0)),
        out_shape=jax.ShapeDtypeStruct((NA, DF), jnp.float32),
    )(h, agg, wua)


def _readout_body(h_ref, sp_ref, w1_ref, w2_ref, w3_ref, eref_ref, out_ref):
    i = pl.program_id(0)

    @pl.when(i == 0)
    def _():
        out_ref[...] = jnp.zeros_like(out_ref)

    hfull = h_ref[...]
    one = jnp.ones_like(hfull[:, :1])
    zpad = jnp.zeros((hfull.shape[0], 127), jnp.float32)
    e = jax.nn.silu(_mm(jnp.concatenate([hfull, one, zpad], axis=1),
                        w1_ref[...]))
    e = jax.nn.silu(_mm(jnp.concatenate([e, one, zpad], axis=1), w2_ref[...]))
    e = _mm(jnp.concatenate([e, one, zpad[:, :63]], axis=1), w3_ref[...])
    sp = sp_ref[...]
    oh = (sp == jnp.arange(NSPEC, dtype=jnp.int32)[None, :]).astype(jnp.float32)
    e = e + _mm(oh, eref_ref[...])
    out_ref[...] += jnp.sum(e).reshape(1, 1)


def _readout_call(h, species2, w1a, w2a, w3a, eref2):
    return pl.pallas_call(
        _readout_body,
        grid=(NA // ATILE,),
        in_specs=[
            pl.BlockSpec((ATILE, DF), lambda i: (i, 0)),
            pl.BlockSpec((ATILE, 1), lambda i: (i, 0)),
            pl.BlockSpec((2 * DF, DF), lambda i: (0, 0)),
            pl.BlockSpec((2 * DF, 64), lambda i: (0, 0)),
            pl.BlockSpec((DF, 1), lambda i: (0, 0)),
            pl.BlockSpec((NSPEC, 1), lambda i: (0, 0)),
        ],
        out_specs=pl.BlockSpec((1, 1), lambda i: (0, 0)),
        out_shape=jax.ShapeDtypeStruct((1, 1), jnp.float32),
    )(h, species2, w1a, w2a, w3a, eref2)


def kernel(species, positions, batch, edge_index, compute_forces,
           species_emb, e_ref, W1, b1, W2, b2, Wsh, Wu, bu,
           he_W1, he_b1, he_W2, he_b2, he_W3, he_b3):
    del batch, compute_forces
    f32 = jnp.float32
    src, dst = edge_index[0], edge_index[1]
    npad = EP - NE
    src_p = jnp.concatenate([src, jnp.zeros((npad,), jnp.int32)])
    dst_p = jnp.concatenate([dst, jnp.full((npad,), NA, jnp.int32)])
    srcr = src_p.reshape(EP // 128, 128)
    dstr = dst_p.reshape(EP // 128, 128)
    pos_pad = jnp.zeros((10240, 16), f32).at[:NA, :3].set(positions)

    w1s = W1.transpose(1, 0, 2).reshape(NBAS, NBLOCK * NHID)
    w1a = (jnp.zeros((128, NBLOCK * NHID), f32)
           .at[:NBAS].set(w1s).at[NBAS].set(b1.reshape(-1)))
    w2a = jnp.zeros((384, NBLOCK * DF), f32)
    for b in range(NBLOCK):
        w2a = w2a.at[b * NHID:(b + 1) * NHID, b * DF:(b + 1) * DF].set(W2[b])
    w2a = w2a.at[NBLOCK * NHID].set(b2.reshape(-1))
    wsha = (jnp.zeros((128, NBLOCK * DF), f32)
            .at[:9].set(Wsh.transpose(1, 0, 2).reshape(9, NBLOCK * DF)))
    wu_s = Wu / AVGNEIGH
    wua = (jnp.zeros((NBLOCK, 2 * DF, DF), f32)
           .at[:, :DF].set(wu_s).at[:, DF].set(bu))
    hw1a = (jnp.zeros((2 * DF, DF), f32)
            .at[:DF].set(he_W1).at[DF].set(he_b1))
    hw2a = (jnp.zeros((2 * DF, 64), f32)
            .at[:DF].set(he_W2).at[DF].set(he_b2))
    hw3a = jnp.zeros((DF, 1), f32).at[:64].set(he_W3).at[64].set(he_b3)
    species2 = species.reshape(NA, 1)
    eref2 = e_ref.reshape(NSPEC, 1)

    ps, pd = _geo_kernel(pos_pad, srcr, dstr)
    gains = _gains_call(ps, pd, w1a, w2a, wsha)
    h = _h0_call(species2, species_emb)
    for b in range(NBLOCK):
        agg = _scatter_kernel(h, gains[b], srcr, dstr)
        h = _upd_call(h, agg.reshape(2, NA, DF), wua[b])
    out = _readout_call(h, species2, hw1a, hw2a, hw3a, eref2)
    return out.reshape((1,))


# speedup vs baseline: 1.3564x; 1.0883x over previous
import functools

import jax
import jax.numpy as jnp
from jax import lax


def _mm(a, b):
    return jax.lax.dot_general(
        a, b, (((1,), (0,)), ((), ())),
        precision=jax.lax.Precision.HIGHEST,
        preferred_element_type=jnp.float32)
from jax.experimental import pallas as pl
from jax.experimental.pallas import tpu as pltpu
from jax.experimental.pallas import tpu_sc as plsc

NA = 10000
NE = 320000
EP = 327680
DF = 128
NBAS = 8
NBLOCK = 4
NHID = 64
CUTOFF = 5.0
AVGNEIGH = 32.0
NSPEC = 100

NCORE = 2
NSUB = 16
NW = NCORE * NSUB

CHUNK = 1024
KSUB = 8
QUART = 128
PER_TILE = EP // NW
NCHUNK = PER_TILE // CHUNK
OUT_ROWS = 632
OUT_ROWS_LAST = NA - 15 * OUT_ROWS
AGG_ROWS = 10016


@functools.cache
def _mesh():
    return plsc.VectorSubcoreMesh(
        core_axis_name="c", subcore_axis_name="s",
        num_cores=NCORE, num_subcores=NSUB)


GEO_CHUNK = 1024
GEO_NCHUNK = PER_TILE // GEO_CHUNK


def _geo_body(posr_hbm, srcr, dstr, ps_out, pd_out, sidx2, didx2, psbuf,
              pdbuf, sem):
    c = lax.axis_index("c")
    s = lax.axis_index("s")
    wid = c * NSUB + s

    def chunk_body(i, carry):
        base = pl.multiple_of(wid * PER_TILE + i * GEO_CHUNK, GEO_CHUNK)
        brow = pl.multiple_of(base // 128, 8)
        pltpu.async_copy(srcr.at[pl.ds(brow, KSUB)], sidx2, sem).wait()
        pltpu.async_copy(dstr.at[pl.ds(brow, KSUB)], didx2, sem).wait()
        descs = []
        for j in range(KSUB):
            descs.append(pltpu.async_copy(
                posr_hbm.at[sidx2.at[j]], psbuf.at[pl.ds(j * 128, 128)], sem))
            descs.append(pltpu.async_copy(
                posr_hbm.at[didx2.at[j]], pdbuf.at[pl.ds(j * 128, 128)], sem))
        for d in descs:
            d.wait()
        pltpu.async_copy(psbuf, ps_out.at[pl.ds(base, GEO_CHUNK)], sem).wait()
        pltpu.async_copy(pdbuf, pd_out.at[pl.ds(base, GEO_CHUNK)], sem).wait()
        return carry

    lax.fori_loop(0, GEO_NCHUNK, chunk_body, 0)


def _geo_kernel(*args):
    return pl.kernel(
        _geo_body,
        out_type=(jax.ShapeDtypeStruct((EP, 16), jnp.float32),
                  jax.ShapeDtypeStruct((EP, 16), jnp.float32)),
        mesh=_mesh(),
        scratch_types=[
            pltpu.VMEM((KSUB, 128), jnp.int32),
            pltpu.VMEM((KSUB, 128), jnp.int32),
            pltpu.VMEM((GEO_CHUNK, 16), jnp.float32),
            pltpu.VMEM((GEO_CHUNK, 16), jnp.float32),
            pltpu.SemaphoreType.DMA,
        ],
        compiler_params=pltpu.CompilerParams(use_tc_tiling_on_sc=False),
    )(*args)


def _scatter_body(h, gh, pk, out, agg_sh, packed, rows0, rows1, gbuf,
                  sem, sem_g, sem_s):
    c = lax.axis_index("c")
    s = lax.axis_index("s")
    wid = c * NSUB + s
    zeros16 = jnp.zeros((16,), jnp.float32)
    rowsbufs = (rows0, rows1)

    def zbody(r, carry):
        for k in range(8):
            rows0[r, pl.ds(k * 16, 16)] = zeros16
        return carry
    lax.fori_loop(0, QUART, zbody, 0)
    z0 = s * OUT_ROWS
    for t in range(4):
        pltpu.sync_copy(rows0, agg_sh.at[pl.ds(z0 + t * QUART, QUART)])

    @pl.when(s < 15)
    def _():
        pltpu.sync_copy(rows0.at[pl.ds(0, 120)],
                        agg_sh.at[pl.ds(z0 + 4 * QUART, 120)])

    @pl.when(s == 15)
    def _():
        pltpu.sync_copy(rows0.at[pl.ds(0, 24)],
                        agg_sh.at[pl.ds(z0 + 4 * QUART, 24)])
    plsc.subcore_barrier()

    def src_vec(q, k):
        v = packed[q, pl.ds(k * 16, 16)]
        return v & 0xFFFF

    def dst_vec(q, k):
        v = packed[q, pl.ds(k * 16, 16)]
        return lax.shift_right_logical(v, 16)

    def issue_gathers(q):
        p = q & 1
        return [pltpu.async_copy(h.at[src_vec(q, k)],
                                 rowsbufs[p].at[pl.ds(k * 16, 16)], sem_g)
                for k in range(8)]

    def drain_scatters():
        pltpu.make_async_copy(gh.at[pl.ds(0, QUART)], gbuf, sem_s).wait()

    def chunk_body(i, carry):
        base = pl.multiple_of(wid * PER_TILE + i * CHUNK, CHUNK)
        brow = pl.multiple_of(base // 128, 8)
        pltpu.async_copy(pk.at[pl.ds(brow, KSUB)], packed, sem).wait()
        gds = issue_gathers(0)
        for q in range(KSUB):
            p = q & 1
            for d in gds:
                d.wait()
            if q == 0:
                @pl.when(i > 0)
                def _():
                    drain_scatters()
            else:
                drain_scatters()
            if q < KSUB - 1:
                gds = issue_gathers(q + 1)
            qb = pl.multiple_of(base + q * QUART, QUART)
            pltpu.sync_copy(gh.at[pl.ds(qb, QUART)], gbuf)

            def mul_body(r, carry2):
                for rr in range(2):
                    for k in range(8):
                        sl = pl.ds(k * 16, 16)
                        rowsbufs[p][2 * r + rr, sl] = (
                            rowsbufs[p][2 * r + rr, sl] * gbuf[2 * r + rr, sl])
                return carry2
            lax.fori_loop(0, QUART // 2, mul_body, 0)

            for k in range(8):
                pltpu.async_copy(rowsbufs[p].at[pl.ds(k * 16, 16)],
                                 agg_sh.at[dst_vec(q, k)], sem_s, add=True)
        return carry

    lax.fori_loop(0, NCHUNK, chunk_body, 0)
    drain_scatters()
    plsc.subcore_barrier()
    o0 = pl.multiple_of(s * OUT_ROWS, 8)
    oo = pl.multiple_of(c * NA + o0, 8)

    @pl.when(s < 15)
    def _():
        pltpu.sync_copy(agg_sh.at[pl.ds(o0, OUT_ROWS)],
                        out.at[pl.ds(oo, OUT_ROWS)])

    @pl.when(s == 15)
    def _():
        pltpu.sync_copy(agg_sh.at[pl.ds(o0, OUT_ROWS_LAST)],
                        out.at[pl.ds(oo, OUT_ROWS_LAST)])


def _scatter_kernel(*args):
    return pl.kernel(
        _scatter_body,
        out_type=jax.ShapeDtypeStruct((2 * NA, DF), jnp.float32),
        mesh=_mesh(),
        scratch_types=[
            pltpu.VMEM_SHARED((AGG_ROWS, DF), jnp.float32),
            pltpu.VMEM((KSUB, 128), jnp.int32),
            pltpu.VMEM((QUART, DF), jnp.float32),
            pltpu.VMEM((QUART, DF), jnp.float32),
            pltpu.VMEM((QUART, DF), jnp.float32),
            pltpu.SemaphoreType.DMA,
            pltpu.SemaphoreType.DMA,
            pltpu.SemaphoreType.DMA,
        ],
    )(*args)


GTILE = 1024


def _gains_body(ps_ref, pd_ref, w1a_ref, w2a_ref, wsha_ref, out_ref):
    ps = ps_ref[...]
    pd = pd_ref[...]
    dx = pd[:, 0:1] - ps[:, 0:1]
    dy = pd[:, 1:2] - ps[:, 1:2]
    dz = pd[:, 2:3] - ps[:, 2:3]
    r2 = dx * dx + dy * dy + dz * dz
    r = jnp.sqrt(r2)
    rinv = 1.0 / (r + 1e-8)
    x = dx * rinv
    y = dy * rinv
    z = dz * rinv
    s3 = jnp.sqrt(3.0)
    s15 = jnp.sqrt(15.0)
    s5 = jnp.sqrt(5.0)
    one = jnp.ones_like(x)
    sh = jnp.concatenate([
        one, s3 * x, s3 * y, s3 * z,
        s15 * x * y, s15 * y * z,
        (s5 / 2.0) * (2.0 * z * z - x * x - y * y),
        s15 * x * z, (s15 / 2.0) * (x * x - y * y),
    ], axis=1)
    r_safe = jnp.where(r > 1e-8, r, 1e-8)
    n = (jnp.arange(NBAS, dtype=jnp.int32) + 1).astype(jnp.float32)[None, :]
    rbf = jnp.sqrt(2.0 / CUTOFF) * jnp.sin(n * (jnp.pi / CUTOFF) * r_safe) / r_safe
    u = r / CUTOFF
    u6 = u * u * u * u * u * u
    env = (1.0 - 28.0 * u6 + 48.0 * u6 * u - 21.0 * u6 * u * u)
    env = env * (u < 1.0).astype(jnp.float32)
    rbf = rbf * env
    zpad = jnp.zeros((rbf.shape[0], 119), jnp.float32)
    rbf_aug = jnp.concatenate([rbf, one, zpad], axis=1)
    hid = jax.nn.silu(_mm(rbf_aug, w1a_ref[...]))
    hid_aug = jnp.concatenate(
        [hid, one, jnp.zeros((rbf.shape[0], 127), jnp.float32)], axis=1)
    w = _mm(hid_aug, w2a_ref[...])
    gate = jax.nn.silu(_mm(jnp.concatenate([sh, zpad], axis=1),
                           wsha_ref[...]))
    g = w * gate
    for b in range(NBLOCK):
        out_ref[b] = g[:, b * DF:(b + 1) * DF]


def _gains_call(ps, pd, w1a, w2a, wsha):
    return pl.pallas_call(
        _gains_body,
        grid=(EP // GTILE,),
        in_specs=[
            pl.BlockSpec((GTILE, 16), lambda i: (i, 0)),
            pl.BlockSpec((GTILE, 16), lambda i: (i, 0)),
            pl.BlockSpec((128, 256), lambda i: (0, 0)),
            pl.BlockSpec((384, 512), lambda i: (0, 0)),
            pl.BlockSpec((128, 512), lambda i: (0, 0)),
        ],
        out_specs=pl.BlockSpec((NBLOCK, GTILE, DF), lambda i: (0, i, 0)),
        out_shape=jax.ShapeDtypeStruct((NBLOCK, EP, DF), jnp.float32),
    )(ps, pd, w1a, w2a, wsha)


ATILE = 2000


def _h0_body(sp_ref, emb_ref, out_ref):
    sp = sp_ref[...]
    oh = (sp == jnp.arange(NSPEC, dtype=jnp.int32)[None, :]).astype(jnp.float32)
    out_ref[...] = _mm(oh, emb_ref[...])


def _h0_call(species2, species_emb):
    return pl.pallas_call(
        _h0_body,
        grid=(NA // ATILE,),
        in_specs=[
            pl.BlockSpec((ATILE, 1), lambda i: (i, 0)),
            pl.BlockSpec((NSPEC, DF), lambda i: (0, 0)),
        ],
        out_specs=pl.BlockSpec((ATILE, DF), lambda i: (i, 0)),
        out_shape=jax.ShapeDtypeStruct((NA, DF), jnp.float32),
    )(species2, species_emb)


def _upd_body(h_ref, agg_ref, wua_ref, out_ref):
    a2 = agg_ref[...]
    afull = a2[0] + a2[1]
    one = jnp.ones_like(afull[:, :1])
    zpad = jnp.zeros((afull.shape[0], 127), jnp.float32)
    aug = jnp.concatenate([afull, one, zpad], axis=1)
    out_ref[...] = h_ref[...] + jax.nn.silu(_mm(aug, wua_ref[...]))


def _upd_call(h, agg, wua):
    return pl.pallas_call(
        _upd_body,
        grid=(NA // ATILE,),
        in_specs=[
            pl.BlockSpec((ATILE, DF), lambda i: (i, 0)),
            pl.BlockSpec((2, ATILE, DF), lambda i: (0, i, 0)),
            pl.BlockSpec((2 * DF, DF), lambda i: (0, 0)),
        ],
        out_specs=pl.BlockSpec((ATILE, DF), lambda i: (i, 0)),
        out_shape=jax.ShapeDtypeStruct((NA, DF), jnp.float32),
    )(h, agg, wua)


def _readout_body(h_ref, sp_ref, w1_ref, w2_ref, w3_ref, eref_ref, out_ref):
    i = pl.program_id(0)

    @pl.when(i == 0)
    def _():
        out_ref[...] = jnp.zeros_like(out_ref)

    hfull = h_ref[...]
    one = jnp.ones_like(hfull[:, :1])
    zpad = jnp.zeros((hfull.shape[0], 127), jnp.float32)
    e = jax.nn.silu(_mm(jnp.concatenate([hfull, one, zpad], axis=1),
                        w1_ref[...]))
    e = jax.nn.silu(_mm(jnp.concatenate([e, one, zpad], axis=1), w2_ref[...]))
    e = _mm(jnp.concatenate([e, one, zpad[:, :63]], axis=1), w3_ref[...])
    sp = sp_ref[...]
    oh = (sp == jnp.arange(NSPEC, dtype=jnp.int32)[None, :]).astype(jnp.float32)
    e = e + _mm(oh, eref_ref[...])
    out_ref[...] += jnp.sum(e).reshape(1, 1)


def _readout_call(h, species2, w1a, w2a, w3a, eref2):
    return pl.pallas_call(
        _readout_body,
        grid=(NA // ATILE,),
        in_specs=[
            pl.BlockSpec((ATILE, DF), lambda i: (i, 0)),
            pl.BlockSpec((ATILE, 1), lambda i: (i, 0)),
            pl.BlockSpec((2 * DF, DF), lambda i: (0, 0)),
            pl.BlockSpec((2 * DF, 64), lambda i: (0, 0)),
            pl.BlockSpec((DF, 1), lambda i: (0, 0)),
            pl.BlockSpec((NSPEC, 1), lambda i: (0, 0)),
        ],
        out_specs=pl.BlockSpec((1, 1), lambda i: (0, 0)),
        out_shape=jax.ShapeDtypeStruct((1, 1), jnp.float32),
    )(h, species2, w1a, w2a, w3a, eref2)


def kernel(species, positions, batch, edge_index, compute_forces,
           species_emb, e_ref, W1, b1, W2, b2, Wsh, Wu, bu,
           he_W1, he_b1, he_W2, he_b2, he_W3, he_b3):
    del batch, compute_forces
    f32 = jnp.float32
    src, dst = edge_index[0], edge_index[1]
    npad = EP - NE
    src_p = jnp.concatenate([src, jnp.zeros((npad,), jnp.int32)])
    dst_p = jnp.concatenate([dst, jnp.full((npad,), NA, jnp.int32)])
    srcr = src_p.reshape(EP // 128, 128)
    dstr = dst_p.reshape(EP // 128, 128)
    pk = ((dst_p << 16) | src_p).reshape(EP // 128, 128)
    pos_pad = jnp.zeros((10240, 16), f32).at[:NA, :3].set(positions)

    w1s = W1.transpose(1, 0, 2).reshape(NBAS, NBLOCK * NHID)
    w1a = (jnp.zeros((128, NBLOCK * NHID), f32)
           .at[:NBAS].set(w1s).at[NBAS].set(b1.reshape(-1)))
    w2a = jnp.zeros((384, NBLOCK * DF), f32)
    for b in range(NBLOCK):
        w2a = w2a.at[b * NHID:(b + 1) * NHID, b * DF:(b + 1) * DF].set(W2[b])
    w2a = w2a.at[NBLOCK * NHID].set(b2.reshape(-1))
    wsha = (jnp.zeros((128, NBLOCK * DF), f32)
            .at[:9].set(Wsh.transpose(1, 0, 2).reshape(9, NBLOCK * DF)))
    wu_s = Wu / AVGNEIGH
    wua = (jnp.zeros((NBLOCK, 2 * DF, DF), f32)
           .at[:, :DF].set(wu_s).at[:, DF].set(bu))
    hw1a = (jnp.zeros((2 * DF, DF), f32)
            .at[:DF].set(he_W1).at[DF].set(he_b1))
    hw2a = (jnp.zeros((2 * DF, 64), f32)
            .at[:DF].set(he_W2).at[DF].set(he_b2))
    hw3a = jnp.zeros((DF, 1), f32).at[:64].set(he_W3).at[64].set(he_b3)
    species2 = species.reshape(NA, 1)
    eref2 = e_ref.reshape(NSPEC, 1)

    ps, pd = _geo_kernel(pos_pad, srcr, dstr)
    gains = _gains_call(ps, pd, w1a, w2a, wsha)
    h = _h0_call(species2, species_emb)
    for b in range(NBLOCK):
        agg = _scatter_kernel(h, gains[b], pk)
        h = _upd_call(h, agg.reshape(2, NA, DF), wua[b])
    out = _readout_call(h, species2, hw1a, hw2a, hw3a, eref2)
    return out.reshape((1,))


# speedup vs baseline: 1.8782x; 1.3847x over previous
import functools

import jax
import jax.numpy as jnp
from jax import lax


def _mm(a, b):
    return jax.lax.dot_general(
        a, b, (((1,), (0,)), ((), ())),
        precision=jax.lax.Precision.HIGHEST,
        preferred_element_type=jnp.float32)


def _mmh(a, b):
    f32, bf = jnp.float32, jnp.bfloat16

    def d(x, y):
        return jax.lax.dot_general(x, y, (((1,), (0,)), ((), ())),
                                   preferred_element_type=f32)
    ah = a.astype(bf)
    al = (a - ah.astype(f32)).astype(bf)
    bh = b.astype(bf)
    bl = (b - bh.astype(f32)).astype(bf)
    return d(ah, bh) + d(al, bh) + d(ah, bl)
from jax.experimental import pallas as pl
from jax.experimental.pallas import tpu as pltpu
from jax.experimental.pallas import tpu_sc as plsc

NA = 10000
NE = 320000
EP = 327680
DF = 128
NBAS = 8
NBLOCK = 4
NHID = 64
CUTOFF = 5.0
AVGNEIGH = 32.0
NSPEC = 100

NCORE = 2
NSUB = 16
NW = NCORE * NSUB

CHUNK = 1024
KSUB = 8
QUART = 128
PER_TILE = EP // NW
NCHUNK = PER_TILE // CHUNK
CH_FAST = 13
CH_SLOW = 20 - CH_FAST
OUT_ROWS = 632
OUT_ROWS_LAST = NA - 15 * OUT_ROWS
AGG_ROWS = 10016


@functools.cache
def _mesh():
    return plsc.VectorSubcoreMesh(
        core_axis_name="c", subcore_axis_name="s",
        num_cores=NCORE, num_subcores=NSUB)


GEO_CHUNK = 1024
GEO_NCHUNK = PER_TILE // GEO_CHUNK


def _geo_body(posr_hbm, srcr, dstr, ps_out, pd_out, sidx2, didx2, psbuf,
              pdbuf, sem):
    c = lax.axis_index("c")
    s = lax.axis_index("s")
    wid = c * NSUB + s

    def chunk_body(i, carry):
        base = pl.multiple_of(wid * PER_TILE + i * GEO_CHUNK, GEO_CHUNK)
        brow = pl.multiple_of(base // 128, 8)
        pltpu.async_copy(srcr.at[pl.ds(brow, KSUB)], sidx2, sem).wait()
        pltpu.async_copy(dstr.at[pl.ds(brow, KSUB)], didx2, sem).wait()
        descs = []
        for j in range(KSUB):
            descs.append(pltpu.async_copy(
                posr_hbm.at[sidx2.at[j]], psbuf.at[pl.ds(j * 128, 128)], sem))
            descs.append(pltpu.async_copy(
                posr_hbm.at[didx2.at[j]], pdbuf.at[pl.ds(j * 128, 128)], sem))
        for d in descs:
            d.wait()
        pltpu.async_copy(psbuf, ps_out.at[pl.ds(base, GEO_CHUNK)], sem).wait()
        pltpu.async_copy(pdbuf, pd_out.at[pl.ds(base, GEO_CHUNK)], sem).wait()
        return carry

    lax.fori_loop(0, GEO_NCHUNK, chunk_body, 0)


def _geo_kernel(*args):
    return pl.kernel(
        _geo_body,
        out_type=(jax.ShapeDtypeStruct((EP, 16), jnp.float32),
                  jax.ShapeDtypeStruct((EP, 16), jnp.float32)),
        mesh=_mesh(),
        scratch_types=[
            pltpu.VMEM((KSUB, 128), jnp.int32),
            pltpu.VMEM((KSUB, 128), jnp.int32),
            pltpu.VMEM((GEO_CHUNK, 16), jnp.float32),
            pltpu.VMEM((GEO_CHUNK, 16), jnp.float32),
            pltpu.SemaphoreType.DMA,
        ],
        compiler_params=pltpu.CompilerParams(use_tc_tiling_on_sc=False),
    )(*args)


def _scatter_body(h, gh, pk, out, agg_sh, packed, rows0, rows1, gbuf,
                  sem, sem_g, sem_s):
    c = lax.axis_index("c")
    s = lax.axis_index("s")
    wid = c * NSUB + s
    zeros16 = jnp.zeros((16,), jnp.float32)
    rowsbufs = (rows0, rows1)

    def zbody(r, carry):
        for k in range(8):
            rows0[r, pl.ds(k * 16, 16)] = zeros16
        return carry
    lax.fori_loop(0, QUART, zbody, 0)
    z0 = s * OUT_ROWS
    for t in range(4):
        pltpu.sync_copy(rows0, agg_sh.at[pl.ds(z0 + t * QUART, QUART)])

    @pl.when(s < 15)
    def _():
        pltpu.sync_copy(rows0.at[pl.ds(0, 120)],
                        agg_sh.at[pl.ds(z0 + 4 * QUART, 120)])

    @pl.when(s == 15)
    def _():
        pltpu.sync_copy(rows0.at[pl.ds(0, 24)],
                        agg_sh.at[pl.ds(z0 + 4 * QUART, 24)])
    plsc.subcore_barrier()

    def src_vec(q, k):
        v = packed[q, pl.ds(k * 16, 16)]
        return v & 0xFFFF

    def dst_vec(q, k):
        v = packed[q, pl.ds(k * 16, 16)]
        return lax.shift_right_logical(v, 16)

    def issue_gathers(q):
        p = q & 1
        return [pltpu.async_copy(h.at[src_vec(q, k)],
                                 rowsbufs[p].at[pl.ds(k * 16, 16)], sem_g)
                for k in range(8)]

    def drain_scatters():
        pltpu.make_async_copy(gh.at[pl.ds(0, QUART)], gbuf, sem_s).wait()

    nch = jnp.where(c == 0, CH_FAST, CH_SLOW)
    tbase = jnp.where(c == 0, s * (CH_FAST * CHUNK),
                      NSUB * CH_FAST * CHUNK + s * (CH_SLOW * CHUNK))

    def chunk_body(i, carry):
        base = pl.multiple_of(tbase + i * CHUNK, CHUNK)
        brow = pl.multiple_of(base // 128, 8)
        pltpu.async_copy(pk.at[pl.ds(brow, KSUB)], packed, sem).wait()
        gds = issue_gathers(0)
        for q in range(KSUB):
            p = q & 1
            for d in gds:
                d.wait()
            if q == 0:
                @pl.when(i > 0)
                def _():
                    drain_scatters()
            else:
                drain_scatters()
            if q < KSUB - 1:
                gds = issue_gathers(q + 1)
            qb = pl.multiple_of(base + q * QUART, QUART)
            pltpu.sync_copy(gh.at[pl.ds(qb, QUART)], gbuf)

            def mul_body(r, carry2):
                for rr in range(2):
                    for k in range(8):
                        sl = pl.ds(k * 16, 16)
                        rowsbufs[p][2 * r + rr, sl] = (
                            rowsbufs[p][2 * r + rr, sl] * gbuf[2 * r + rr, sl])
                return carry2
            lax.fori_loop(0, QUART // 2, mul_body, 0)

            for k in range(8):
                pltpu.async_copy(rowsbufs[p].at[pl.ds(k * 16, 16)],
                                 agg_sh.at[dst_vec(q, k)], sem_s, add=True)
        return carry

    lax.fori_loop(0, nch, chunk_body, 0)
    drain_scatters()
    plsc.subcore_barrier()
    o0 = pl.multiple_of(s * OUT_ROWS, 8)
    oo = pl.multiple_of(c * NA + o0, 8)

    @pl.when(s < 15)
    def _():
        pltpu.sync_copy(agg_sh.at[pl.ds(o0, OUT_ROWS)],
                        out.at[pl.ds(oo, OUT_ROWS)])

    @pl.when(s == 15)
    def _():
        pltpu.sync_copy(agg_sh.at[pl.ds(o0, OUT_ROWS_LAST)],
                        out.at[pl.ds(oo, OUT_ROWS_LAST)])


def _scatter_kernel(*args):
    return pl.kernel(
        _scatter_body,
        out_type=jax.ShapeDtypeStruct((2 * NA, DF), jnp.float32),
        mesh=_mesh(),
        scratch_types=[
            pltpu.VMEM_SHARED((AGG_ROWS, DF), jnp.float32),
            pltpu.VMEM((KSUB, 128), jnp.int32),
            pltpu.VMEM((QUART, DF), jnp.float32),
            pltpu.VMEM((QUART, DF), jnp.float32),
            pltpu.VMEM((QUART, DF), jnp.float32),
            pltpu.SemaphoreType.DMA,
            pltpu.SemaphoreType.DMA,
            pltpu.SemaphoreType.DMA,
        ],
    )(*args)


GTILE = 1024


def _gains_body(ps_ref, pd_ref, w1a_ref, w2s_ref, wsha_ref,
                o0_ref, o1_ref, o2_ref, o3_ref):
    ps = ps_ref[...]
    pd = pd_ref[...]
    dx = pd[:, 0:1] - ps[:, 0:1]
    dy = pd[:, 1:2] - ps[:, 1:2]
    dz = pd[:, 2:3] - ps[:, 2:3]
    r2 = dx * dx + dy * dy + dz * dz
    r = jnp.sqrt(r2)
    rinv = 1.0 / (r + 1e-8)
    x = dx * rinv
    y = dy * rinv
    z = dz * rinv
    s3 = jnp.sqrt(3.0)
    s15 = jnp.sqrt(15.0)
    s5 = jnp.sqrt(5.0)
    one = jnp.ones_like(x)
    sh = jnp.concatenate([
        one, s3 * x, s3 * y, s3 * z,
        s15 * x * y, s15 * y * z,
        (s5 / 2.0) * (2.0 * z * z - x * x - y * y),
        s15 * x * z, (s15 / 2.0) * (x * x - y * y),
    ], axis=1)
    r_safe = jnp.where(r > 1e-8, r, 1e-8)
    n = (jnp.arange(NBAS, dtype=jnp.int32) + 1).astype(jnp.float32)[None, :]
    rbf = jnp.sqrt(2.0 / CUTOFF) * jnp.sin(n * (jnp.pi / CUTOFF) * r_safe) / r_safe
    u = r / CUTOFF
    u6 = u * u * u * u * u * u
    env = (1.0 - 28.0 * u6 + 48.0 * u6 * u - 21.0 * u6 * u * u)
    env = env * (u < 1.0).astype(jnp.float32)
    rbf = rbf * env
    zpad = jnp.zeros((rbf.shape[0], 119), jnp.float32)
    rbf_aug = jnp.concatenate([rbf, one, zpad], axis=1)
    hid = jax.nn.silu(_mmh(rbf_aug, w1a_ref[...]))
    gate = jax.nn.silu(_mmh(jnp.concatenate([sh, zpad], axis=1),
                            wsha_ref[...]))
    zpad63 = jnp.zeros((rbf.shape[0], 63), jnp.float32)
    w2all = w2s_ref[...]
    outs = (o0_ref, o1_ref, o2_ref, o3_ref)
    for b in range(NBLOCK):
        hb = jnp.concatenate(
            [hid[:, b * NHID:(b + 1) * NHID], one, zpad63], axis=1)
        w = _mmh(hb, w2all[b * DF:(b + 1) * DF])
        outs[b][...] = w * gate[:, b * DF:(b + 1) * DF]


def _gains_call(ps, pd, w1a, w2s, wsha):
    shp = jax.ShapeDtypeStruct((EP, DF), jnp.float32)
    return pl.pallas_call(
        _gains_body,
        grid=(EP // GTILE,),
        in_specs=[
            pl.BlockSpec((GTILE, 16), lambda i: (i, 0)),
            pl.BlockSpec((GTILE, 16), lambda i: (i, 0)),
            pl.BlockSpec((128, 256), lambda i: (0, 0)),
            pl.BlockSpec((512, 128), lambda i: (0, 0)),
            pl.BlockSpec((128, 512), lambda i: (0, 0)),
        ],
        out_specs=[pl.BlockSpec((GTILE, DF), lambda i: (i, 0))
                   for _ in range(NBLOCK)],
        out_shape=[shp] * NBLOCK,
    )(ps, pd, w1a, w2s, wsha)


ATILE = 2000


def _h0_body(sp_ref, emb_ref, out_ref):
    sp = sp_ref[...]
    oh = (sp == jnp.arange(NSPEC, dtype=jnp.int32)[None, :]).astype(jnp.float32)
    out_ref[...] = _mm(oh, emb_ref[...])


def _h0_call(species2, species_emb):
    return pl.pallas_call(
        _h0_body,
        grid=(NA // ATILE,),
        in_specs=[
            pl.BlockSpec((ATILE, 1), lambda i: (i, 0)),
            pl.BlockSpec((NSPEC, DF), lambda i: (0, 0)),
        ],
        out_specs=pl.BlockSpec((ATILE, DF), lambda i: (i, 0)),
        out_shape=jax.ShapeDtypeStruct((NA, DF), jnp.float32),
    )(species2, species_emb)


def _upd_body(h_ref, agg_ref, wua_ref, out_ref):
    a2 = agg_ref[...]
    afull = a2[0] + a2[1]
    one = jnp.ones_like(afull[:, :1])
    zpad = jnp.zeros((afull.shape[0], 127), jnp.float32)
    aug = jnp.concatenate([afull, one, zpad], axis=1)
    out_ref[...] = h_ref[...] + jax.nn.silu(_mmh(aug, wua_ref[...]))


def _upd_call(h, agg, wua):
    return pl.pallas_call(
        _upd_body,
        grid=(NA // ATILE,),
        in_specs=[
            pl.BlockSpec((ATILE, DF), lambda i: (i, 0)),
            pl.BlockSpec((2, ATILE, DF), lambda i: (0, i, 0)),
            pl.BlockSpec((2 * DF, DF), lambda i: (0, 0)),
        ],
        out_specs=pl.BlockSpec((ATILE, DF), lambda i: (i, 0)),
        out_shape=jax.ShapeDtypeStruct((NA, DF), jnp.float32),
    )(h, agg, wua)


def _readout_body(h_ref, sp_ref, w1_ref, w2_ref, w3_ref, eref_ref, out_ref):
    i = pl.program_id(0)

    @pl.when(i == 0)
    def _():
        out_ref[...] = jnp.zeros_like(out_ref)

    hfull = h_ref[...]
    one = jnp.ones_like(hfull[:, :1])
    zpad = jnp.zeros((hfull.shape[0], 127), jnp.float32)
    e = jax.nn.silu(_mmh(jnp.concatenate([hfull, one, zpad], axis=1),
                         w1_ref[...]))
    e = jax.nn.silu(_mmh(jnp.concatenate([e, one, zpad], axis=1), w2_ref[...]))
    e = _mmh(jnp.concatenate([e, one, zpad[:, :63]], axis=1), w3_ref[...])
    sp = sp_ref[...]
    oh = (sp == jnp.arange(NSPEC, dtype=jnp.int32)[None, :]).astype(jnp.float32)
    e = e + _mm(oh, eref_ref[...])
    out_ref[...] += jnp.sum(e).reshape(1, 1)


def _readout_call(h, species2, w1a, w2a, w3a, eref2):
    return pl.pallas_call(
        _readout_body,
        grid=(NA // ATILE,),
        in_specs=[
            pl.BlockSpec((ATILE, DF), lambda i: (i, 0)),
            pl.BlockSpec((ATILE, 1), lambda i: (i, 0)),
            pl.BlockSpec((2 * DF, DF), lambda i: (0, 0)),
            pl.BlockSpec((2 * DF, 64), lambda i: (0, 0)),
            pl.BlockSpec((DF, 1), lambda i: (0, 0)),
            pl.BlockSpec((NSPEC, 1), lambda i: (0, 0)),
        ],
        out_specs=pl.BlockSpec((1, 1), lambda i: (0, 0)),
        out_shape=jax.ShapeDtypeStruct((1, 1), jnp.float32),
    )(h, species2, w1a, w2a, w3a, eref2)


def kernel(species, positions, batch, edge_index, compute_forces,
           species_emb, e_ref, W1, b1, W2, b2, Wsh, Wu, bu,
           he_W1, he_b1, he_W2, he_b2, he_W3, he_b3):
    del batch, compute_forces
    f32 = jnp.float32
    src, dst = edge_index[0], edge_index[1]
    npad = EP - NE
    src_p = jnp.concatenate([src, jnp.zeros((npad,), jnp.int32)])
    dst_p = jnp.concatenate([dst, jnp.full((npad,), NA, jnp.int32)])
    srcr = src_p.reshape(EP // 128, 128)
    dstr = dst_p.reshape(EP // 128, 128)
    pk = ((dst_p << 16) | src_p).reshape(EP // 128, 128)
    pos_pad = jnp.zeros((10240, 16), f32).at[:NA, :3].set(positions)

    w1s = W1.transpose(1, 0, 2).reshape(NBAS, NBLOCK * NHID)
    w1a = (jnp.zeros((128, NBLOCK * NHID), f32)
           .at[:NBAS].set(w1s).at[NBAS].set(b1.reshape(-1)))
    w2s = jnp.zeros((NBLOCK, DF, DF), f32).at[:, :NHID].set(W2)
    w2s = w2s.at[:, NHID].set(b2).reshape(NBLOCK * DF, DF)
    wsha = (jnp.zeros((128, NBLOCK * DF), f32)
            .at[:9].set(Wsh.transpose(1, 0, 2).reshape(9, NBLOCK * DF)))
    wu_s = Wu / AVGNEIGH
    wua = (jnp.zeros((NBLOCK, 2 * DF, DF), f32)
           .at[:, :DF].set(wu_s).at[:, DF].set(bu))
    hw1a = (jnp.zeros((2 * DF, DF), f32)
            .at[:DF].set(he_W1).at[DF].set(he_b1))
    hw2a = (jnp.zeros((2 * DF, 64), f32)
            .at[:DF].set(he_W2).at[DF].set(he_b2))
    hw3a = jnp.zeros((DF, 1), f32).at[:64].set(he_W3).at[64].set(he_b3)
    species2 = species.reshape(NA, 1)
    eref2 = e_ref.reshape(NSPEC, 1)

    ps, pd = _geo_kernel(pos_pad, srcr, dstr)
    gains = _gains_call(ps, pd, w1a, w2s, wsha)
    h = _h0_call(species2, species_emb)
    for b in range(NBLOCK):
        agg = _scatter_kernel(h, gains[b], pk)
        h = _upd_call(h, agg.reshape(2, NA, DF), wua[b])
    out = _readout_call(h, species2, hw1a, hw2a, hw3a, eref2)
    return out.reshape((1,))
